# baseline (speedup 1.0000x reference)
import functools

import jax
import jax.numpy as jnp
from jax import lax
from jax.experimental import pallas as pl
from jax.experimental.pallas import tpu as pltpu
from jax.experimental.pallas import tpu_sc as plsc

B, C, N, K = 8, 64, 4096, 20
KPAD = 32
TN = 512
A = 5


def _topk_body(xf_ref, q_ref, idx_ref):
    xf = xf_ref[0]
    q = q_ref[0]
    inner = -2.0 * lax.dot_general(
        q, xf, (((0,), (0,)), ((), ())),
        preferred_element_type=jnp.float32,
    )
    xx = jnp.sum(xf * xf, axis=0)[None, :]
    qq = jnp.sum(q * q, axis=0)[:, None]
    dist = (-xx) - inner - qq

    neginf = jnp.float32(-jnp.inf)

    SR = N // 128
    d3 = dist.reshape(TN, SR, 128)
    srow = lax.broadcasted_iota(jnp.int32, (TN, SR, 128), 1)
    vals, idxs = [], []
    for _ in range(A):
        m = jnp.max(d3, axis=1)
        cand = jnp.where(d3 == m[:, None, :], srow, jnp.int32(SR))
        ci = jnp.min(cand, axis=1)
        d3 = jnp.where(srow == ci[:, None, :], neginf, d3)
        vals.append(m)
        idxs.append(ci)
    lane128 = lax.broadcasted_iota(jnp.int32, (TN, 128), 1)
    val_pool = jnp.concatenate(vals, axis=1)
    idx_pool = jnp.concatenate([i * 128 + lane128 for i in idxs], axis=1)

    lane = lax.broadcasted_iota(jnp.int32, (TN, KPAD), 1)

    def step(kk, carry):
        vp, acc = carry
        m = jnp.max(vp, axis=1, keepdims=True)
        eq = vp == m
        sel = jnp.min(jnp.where(eq, idx_pool, jnp.int32(N)),
                      axis=1, keepdims=True)
        acc = jnp.where(lane == kk, sel, acc)
        vp = jnp.where(eq & (idx_pool == sel), neginf, vp)
        return vp, acc

    _, acc = lax.fori_loop(
        0, K, step, (val_pool, jnp.zeros((TN, KPAD), jnp.int32)))
    idx_ref[0] = acc


def _topk_idx(x):
    return pl.pallas_call(
        _topk_body,
        grid=(B, N // TN),
        in_specs=[
            pl.BlockSpec((1, C, N), lambda b, i: (b, 0, 0)),
            pl.BlockSpec((1, C, TN), lambda b, i: (b, 0, i)),
        ],
        out_specs=pl.BlockSpec((1, TN, KPAD), lambda b, i: (b, i, 0)),
        out_shape=jax.ShapeDtypeStruct((B, N, KPAD), jnp.int32),
    )(x, x)


NC = 2
NS = 16
NW = NC * NS
WPB = NW // B
RPW = N // WPB
JSPAN = RPW * K
GROUPS = JSPAN // 16
UNROLL = 8


def _sc_body(xflat, idxf, out, idx_v, xrow_a, xrow_b, diff_a, ctr_a,
             diff_b, ctr_b):
    wid = lax.axis_index("s") * NC + lax.axis_index("c")
    b = wid // WPB
    n0 = (wid % WPB) * RPW
    j0 = (wid % WPB) * JSPAN
    iota16 = lax.iota(jnp.int32, 16)
    PAIRS = C // 2

    pltpu.sync_copy(idxf.at[b, pl.ds(j0, JSPAN)], idx_v)

    def gather_into(xrow, diff_v, ctr_v):
        def g_loop(g, carry2):
            base = g * 16 * UNROLL
            for u in range(UNROLL):
                off = base + u * 16
                ids = idx_v[pl.ds(off, 16)]
                uu = off + iota16
                reps = n0 + (((uu >> 2) * 52429) >> 18)
                nb = plsc.load_gather(xrow, [ids])
                ct = plsc.load_gather(xrow, [reps])
                diff_v[pl.ds(off, 16)] = nb - ct
                ctr_v[pl.ds(off, 16)] = ct
            return carry2

        lax.fori_loop(0, GROUPS // UNROLL, g_loop, 0)

    def half(c, xrow, diff_v, ctr_v):
        pltpu.sync_copy(xflat.at[b * C + c, :], xrow)
        gather_into(xrow, diff_v, ctr_v)
        pltpu.sync_copy(diff_v, out.at[b * 2 * C + c, pl.ds(j0, JSPAN)])
        pltpu.sync_copy(ctr_v, out.at[b * 2 * C + C + c, pl.ds(j0, JSPAN)])

    def cc_loop(cc, carry):
        half(2 * cc, xrow_a, diff_a, ctr_a)
        half(2 * cc + 1, xrow_b, diff_b, ctr_b)
        return carry

    lax.fori_loop(0, PAIRS, cc_loop, 0)


@functools.cache
def _sc_gather():
    return pl.kernel(
        _sc_body,
        out_type=jax.ShapeDtypeStruct((B * 2 * C, N * K), jnp.float32),
        mesh=plsc.VectorSubcoreMesh(core_axis_name="c", subcore_axis_name="s"),
        compiler_params=pltpu.CompilerParams(needs_layout_passes=False),
        scratch_types=[
            pltpu.VMEM((JSPAN,), jnp.int32),
            pltpu.VMEM((N,), jnp.float32),
            pltpu.VMEM((N,), jnp.float32),
            pltpu.VMEM((JSPAN,), jnp.float32),
            pltpu.VMEM((JSPAN,), jnp.float32),
            pltpu.VMEM((JSPAN,), jnp.float32),
            pltpu.VMEM((JSPAN,), jnp.float32),
        ],
    )


def kernel(x, k):
    idx32 = _topk_idx(x)
    shift = jnp.asarray(k, jnp.int32) - K
    idxf = idx32[:, :, :K].reshape(B, N * K) + shift
    idxf = jnp.clip(idxf, 0, N - 1)
    out = _sc_gather()(x.reshape(B * C, N), idxf)
    return out.reshape(B, 2 * C, N, K)


# speedup vs baseline: 1.1122x; 1.1122x over previous
import functools

import jax
import jax.numpy as jnp
from jax import lax
from jax.experimental import pallas as pl
from jax.experimental.pallas import tpu as pltpu
from jax.experimental.pallas import tpu_sc as plsc

B, C, N, K = 8, 64, 4096, 20
KPAD = 32
TN = 512
A = 5


def _topk_body(xf_ref, q_ref, idx_ref):
    xf = xf_ref[0]
    q = q_ref[0]
    inner = -2.0 * lax.dot_general(
        q, xf, (((0,), (0,)), ((), ())),
        preferred_element_type=jnp.float32,
    )
    xx = jnp.sum(xf * xf, axis=0)[None, :]
    qq = jnp.sum(q * q, axis=0)[:, None]
    dist = (-xx) - inner - qq

    neginf = jnp.float32(-jnp.inf)

    SR = N // 128
    d3 = dist.reshape(TN, SR, 128)
    srow = lax.broadcasted_iota(jnp.int32, (TN, SR, 128), 1)
    vals, idxs = [], []
    for _ in range(A):
        m = jnp.max(d3, axis=1)
        cand = jnp.where(d3 == m[:, None, :], srow, jnp.int32(SR))
        ci = jnp.min(cand, axis=1)
        d3 = jnp.where(srow == ci[:, None, :], neginf, d3)
        vals.append(m)
        idxs.append(ci)
    lane128 = lax.broadcasted_iota(jnp.int32, (TN, 128), 1)
    val_pool = jnp.concatenate(vals, axis=1)
    idx_pool = jnp.concatenate([i * 128 + lane128 for i in idxs], axis=1)

    lane = lax.broadcasted_iota(jnp.int32, (TN, KPAD), 1)

    def step(kk, carry):
        vp, acc = carry
        m = jnp.max(vp, axis=1, keepdims=True)
        eq = vp == m
        sel = jnp.min(jnp.where(eq, idx_pool, jnp.int32(N)),
                      axis=1, keepdims=True)
        acc = jnp.where(lane == kk, sel, acc)
        vp = jnp.where(eq & (idx_pool == sel), neginf, vp)
        return vp, acc

    _, acc = lax.fori_loop(
        0, K, step, (val_pool, jnp.zeros((TN, KPAD), jnp.int32)))
    idx_ref[0] = acc


def _topk_idx(x):
    return pl.pallas_call(
        _topk_body,
        grid=(B, N // TN),
        in_specs=[
            pl.BlockSpec((1, C, N), lambda b, i: (b, 0, 0)),
            pl.BlockSpec((1, C, TN), lambda b, i: (b, 0, i)),
        ],
        out_specs=pl.BlockSpec((1, TN, KPAD), lambda b, i: (b, i, 0)),
        out_shape=jax.ShapeDtypeStruct((B, N, KPAD), jnp.int32),
    )(x, x)


NC = 2
NS = 16
NW = NC * NS
WPB = NW // B
RPW = N // WPB
JSPAN = RPW * K
GROUPS = JSPAN // 16
UNROLL = 8


def _sc_body(xflat, idxf, rep, out, idx_v, rep_v, xrow_v, diff_v, ctr_v):
    wid = lax.axis_index("s") * NC + lax.axis_index("c")
    b = wid // WPB
    j0 = (wid % WPB) * JSPAN

    pltpu.sync_copy(idxf.at[b, pl.ds(j0, JSPAN)], idx_v)
    pltpu.sync_copy(rep.at[pl.ds(j0, JSPAN)], rep_v)

    def c_loop(c, carry):
        pltpu.sync_copy(xflat.at[b * C + c, :], xrow_v)

        def g_loop(g, carry2):
            base = g * 16 * UNROLL
            for u in range(UNROLL):
                off = base + u * 16
                ids = idx_v[pl.ds(off, 16)]
                reps = rep_v[pl.ds(off, 16)]
                nb = plsc.load_gather(xrow_v, [ids])
                ct = plsc.load_gather(xrow_v, [reps])
                diff_v[pl.ds(off, 16)] = nb - ct
                ctr_v[pl.ds(off, 16)] = ct
            return carry2

        lax.fori_loop(0, GROUPS // UNROLL, g_loop, 0)
        pltpu.sync_copy(diff_v, out.at[b * 2 * C + c, pl.ds(j0, JSPAN)])
        pltpu.sync_copy(ctr_v, out.at[b * 2 * C + C + c, pl.ds(j0, JSPAN)])
        return carry

    lax.fori_loop(0, C, c_loop, 0)


@functools.cache
def _sc_gather():
    return pl.kernel(
        _sc_body,
        out_type=jax.ShapeDtypeStruct((B * 2 * C, N * K), jnp.float32),
        mesh=plsc.VectorSubcoreMesh(core_axis_name="c", subcore_axis_name="s"),
        compiler_params=pltpu.CompilerParams(needs_layout_passes=False),
        scratch_types=[
            pltpu.VMEM((JSPAN,), jnp.int32),
            pltpu.VMEM((JSPAN,), jnp.int32),
            pltpu.VMEM((N,), jnp.float32),
            pltpu.VMEM((JSPAN,), jnp.float32),
            pltpu.VMEM((JSPAN,), jnp.float32),
        ],
    )


def kernel(x, k):
    idx32 = _topk_idx(x)
    shift = jnp.asarray(k, jnp.int32) - K
    idxf = idx32[:, :, :K].reshape(B, N * K) + shift
    idxf = jnp.clip(idxf, 0, N - 1)
    rep = jnp.arange(N * K, dtype=jnp.int32) // K
    out = _sc_gather()(x.reshape(B * C, N), idxf, rep)
    return out.reshape(B, 2 * C, N, K)
